# Initial kernel scaffold; baseline (speedup 1.0000x reference)
import jax
import jax.numpy as jnp
from jax.experimental import pallas as pl


def kernel(xyz, bound, table, W0, W1):
    raise NotImplementedError("write your pallas kernel here")


# speedup vs baseline: 1.7944x; 1.7944x over previous
import functools
import numpy as np
import jax
import jax.numpy as jnp
from jax import lax
from jax.experimental import pallas as pl
from jax.experimental.pallas import tpu as pltpu
from jax.experimental.pallas import tpu_sc as plsc

_NUM_LEVELS = 16
_LOG2_HASH = 19
_HASH_SIZE = 2 ** _LOG2_HASH
_MASK = _HASH_SIZE - 1
_BASE_RES = 16
_DESIRED = 513
_GRID_RES = 512
_PLS = 2.0 ** (np.log2(_DESIRED / _BASE_RES) / (_NUM_LEVELS - 1))

_OFFS = [0]
_SCALES = []
_RESS = []
for _l in range(_NUM_LEVELS):
    _scale = _BASE_RES * (_PLS ** _l) - 1.0
    _res = int(np.ceil(_scale)) + 1
    _p = min(_HASH_SIZE, _res ** 3)
    _p = (_p + 7) // 8 * 8
    _OFFS.append(_OFFS[-1] + _p)
    _SCALES.append(_scale)
    _RESS.append(_res)
_TOTAL = _OFFS[-1]

_K1 = int(np.uint32(2654435761).astype(np.int32))
_K2 = int(np.uint32(805459861).astype(np.int32))

_NC = 2
_NS = 16
_NW = _NC * _NS
_LANES = 16
_CH = 128
_NSTREAM = 128


def _sc_encode(corners, table_flat, M):
    per_w = M // _NW
    n_chunks = per_w // _CH
    mesh = plsc.VectorSubcoreMesh(core_axis_name="c", subcore_axis_name="s")

    @functools.partial(
        pl.kernel,
        out_type=jax.ShapeDtypeStruct((2 * _NUM_LEVELS, M), jnp.float32),
        mesh=mesh,
        scratch_types=[
            pltpu.VMEM((3, _CH), jnp.int32),
            pltpu.VMEM((_NSTREAM, 128), jnp.int32),
            pltpu.VMEM((_NSTREAM, 128), jnp.int32),
            pltpu.VMEM((_NSTREAM, 128), jnp.float32),
            pltpu.VMEM((_NSTREAM, 128), jnp.float32),
            pltpu.VMEM((_NSTREAM, 128), jnp.float32),
            pltpu.VMEM((2 * _NUM_LEVELS, _CH), jnp.float32),
            pltpu.SemaphoreType.DMA,
        ],
    )
    def enc(corners_hbm, table_hbm, out_hbm,
            cv, idx0b, idx1b, wgtb, rows0, rows1, featb, sem):
        wid = lax.axis_index("s") * _NC + lax.axis_index("c")
        f32 = jnp.float32
        i32 = jnp.int32

        def chunk_body(ci, carry):
            base = wid * per_w + ci * _CH
            pltpu.sync_copy(corners_hbm.at[:, pl.ds(base, _CH)], cv)

            def group_body(g, carry2):
                px = cv[0, pl.ds(g * _LANES, _LANES)]
                py = cv[1, pl.ds(g * _LANES, _LANES)]
                pz = cv[2, pl.ds(g * _LANES, _LANES)]
                fpx = px.astype(f32)
                fpy = py.astype(f32)
                fpz = pz.astype(f32)
                for l in range(_NUM_LEVELS):
                    s_idx = g * _NUM_LEVELS + l
                    sc = float(_SCALES[l] / 512.0)
                    res = _RESS[l]
                    off = _OFFS[l]
                    hashed = res ** 3 > _HASH_SIZE
                    posx = fpx * sc + 0.5
                    posy = fpy * sc + 0.5
                    posz = fpz * sc + 0.5
                    pgx = posx.astype(i32)
                    pgy = posy.astype(i32)
                    pgz = posz.astype(i32)
                    frx = posx - pgx.astype(f32)
                    fry = posy - pgy.astype(f32)
                    frz = posz - pgz.astype(f32)
                    gx = 1.0 - frx
                    gy = 1.0 - fry
                    gz = 1.0 - frz
                    w00 = gy * gz
                    w10 = fry * gz
                    w01 = gy * frz
                    w11 = fry * frz
                    if hashed:
                        ty0 = pgy * _K1
                        ty1 = ty0 + _K1
                        tz0 = pgz * _K2
                        tz1 = tz0 + _K2
                        x0 = pgx
                        x1 = pgx + 1
                    else:
                        r2 = res * res
                        x0 = jnp.minimum(pgx, res - 1)
                        x1 = jnp.minimum(pgx + 1, res - 1)
                        ty0 = jnp.minimum(pgy, res - 1) * res
                        ty1 = jnp.minimum(pgy + 1, res - 1) * res
                        tz0 = jnp.minimum(pgz, res - 1) * r2
                        tz1 = jnp.minimum(pgz + 1, res - 1) * r2
                    for d in range(8):
                        dx, dy, dz = d & 1, (d >> 1) & 1, (d >> 2) & 1
                        xa = x1 if dx else x0
                        ya = ty1 if dy else ty0
                        za = tz1 if dz else tz0
                        if hashed:
                            ridx = ((xa ^ ya ^ za) & _MASK) + off
                        else:
                            ridx = xa + ya + za + off
                        e0 = ridx + ridx
                        wx = frx if dx else gx
                        wyz = (w11 if dz else w10) if dy else (w01 if dz else w00)
                        wv = wx * wyz
                        idx0b[s_idx, pl.ds(d * _LANES, _LANES)] = e0
                        idx1b[s_idx, pl.ds(d * _LANES, _LANES)] = e0 + 1
                        wgtb[s_idx, pl.ds(d * _LANES, _LANES)] = wv
                    pltpu.async_copy(
                        table_hbm.at[idx0b.at[s_idx]], rows0.at[s_idx], sem)
                    pltpu.async_copy(
                        table_hbm.at[idx1b.at[s_idx]], rows1.at[s_idx], sem)
                return carry2

            lax.fori_loop(0, _CH // _LANES, group_body, 0)

            def drain_body(s, carry2):
                pltpu.make_async_copy(
                    table_hbm.at[idx0b.at[s]], rows0.at[s], sem).wait()
                pltpu.make_async_copy(
                    table_hbm.at[idx1b.at[s]], rows1.at[s], sem).wait()
                return carry2

            lax.fori_loop(0, _NSTREAM, drain_body, 0)

            def acc_body(s, carry2):
                g = s // _NUM_LEVELS
                l = s - g * _NUM_LEVELS
                acc0 = jnp.zeros((_LANES,), f32)
                acc1 = jnp.zeros((_LANES,), f32)
                for d in range(8):
                    wv = wgtb[s, pl.ds(d * _LANES, _LANES)]
                    r0 = rows0[s, pl.ds(d * _LANES, _LANES)]
                    r1 = rows1[s, pl.ds(d * _LANES, _LANES)]
                    acc0 = acc0 + wv * r0
                    acc1 = acc1 + wv * r1
                featb[2 * l, pl.ds(g * _LANES, _LANES)] = acc0
                featb[2 * l + 1, pl.ds(g * _LANES, _LANES)] = acc1
                return carry2

            lax.fori_loop(0, _NSTREAM, acc_body, 0)
            pltpu.sync_copy(featb, out_hbm.at[:, pl.ds(base, _CH)])
            return carry

        lax.fori_loop(0, n_chunks, chunk_body, 0)

    return enc(corners, table_flat)


def _tc_mlp(feats, wk, W0, W1, N):
    Bn = 1024
    grid = (N // Bn,)

    def body(f_ref, w_ref, w0_ref, w1_ref, o_ref):
        x = f_ref[...].reshape(2 * _NUM_LEVELS, 8 * Bn)
        h = jnp.maximum(
            jax.lax.dot_general(w0_ref[...], x, (((0,), (0,)), ((), ())),
                                preferred_element_type=jnp.float32), 0.0)
        y = jax.lax.dot_general(w1_ref[...], h, (((0,), (0,)), ((), ())),
                                preferred_element_type=jnp.float32)
        y3 = y.reshape(8, 8, Bn)
        wv = w_ref[...]
        acc = wv[0][None, :] * y3[:, 0, :]
        for k in range(1, 8):
            acc = acc + wv[k][None, :] * y3[:, k, :]
        o_ref[...] = acc.T

    return pl.pallas_call(
        body,
        grid=grid,
        in_specs=[
            pl.BlockSpec((2 * _NUM_LEVELS, 8, Bn), lambda i: (0, 0, i)),
            pl.BlockSpec((8, Bn), lambda i: (0, i)),
            pl.BlockSpec((2 * _NUM_LEVELS, 64), lambda i: (0, 0)),
            pl.BlockSpec((64, 8), lambda i: (0, 0)),
        ],
        out_specs=pl.BlockSpec((Bn, 8), lambda i: (i, 0)),
        out_shape=jax.ShapeDtypeStruct((N, 8), jnp.float32),
    )(feats.reshape(2 * _NUM_LEVELS, 8, N), wk, W0, W1)


def kernel(xyz, bound, table, W0, W1):
    N = xyz.shape[0]
    b = jnp.float32(bound)
    x = (xyz + b) / (2.0 * b)
    coords = x * float(_GRID_RES)
    c0 = jnp.clip(jnp.floor(coords), 0, _GRID_RES - 1).astype(jnp.int32)
    c1 = c0 + 1
    frac = coords - c0.astype(jnp.float32)
    u, v, w = frac[:, 0], frac[:, 1], frac[:, 2]
    kb = np.arange(8)
    kx = (kb & 1).astype(bool)[:, None]
    ky = ((kb >> 1) & 1).astype(bool)[:, None]
    kz = ((kb >> 2) & 1).astype(bool)[:, None]
    cx = jnp.where(kx, c1[:, 0], c0[:, 0]).reshape(-1)
    cy = jnp.where(ky, c1[:, 1], c0[:, 1]).reshape(-1)
    cz = jnp.where(kz, c1[:, 2], c0[:, 2]).reshape(-1)
    corners = jnp.stack([cx, cy, cz], axis=0)
    wk = (jnp.where(kx, u, 1 - u) * jnp.where(ky, v, 1 - v)
          * jnp.where(kz, w, 1 - w)).astype(jnp.float32)
    M = 8 * N
    feats = _sc_encode(corners, table.reshape(-1), M)
    return _tc_mlp(feats, wk, W0, W1, N)


# speedup vs baseline: 5.5580x; 3.0974x over previous
import functools
import numpy as np
import jax
import jax.numpy as jnp
from jax import lax
from jax.experimental import pallas as pl
from jax.experimental.pallas import tpu as pltpu
from jax.experimental.pallas import tpu_sc as plsc

_NUM_LEVELS = 16
_LOG2_HASH = 19
_HASH_SIZE = 2 ** _LOG2_HASH
_MASK = _HASH_SIZE - 1
_BASE_RES = 16
_DESIRED = 513
_GRID_RES = 512
_PLS = 2.0 ** (np.log2(_DESIRED / _BASE_RES) / (_NUM_LEVELS - 1))

_OFFS = [0]
_SCALES = []
_RESS = []
for _l in range(_NUM_LEVELS):
    _scale = _BASE_RES * (_PLS ** _l) - 1.0
    _res = int(np.ceil(_scale)) + 1
    _p = min(_HASH_SIZE, _res ** 3)
    _p = (_p + 7) // 8 * 8
    _OFFS.append(_OFFS[-1] + _p)
    _SCALES.append(_scale)
    _RESS.append(_res)
_TOTAL = _OFFS[-1]

_K1 = int(np.uint32(2654435761).astype(np.int32))
_K2 = int(np.uint32(805459861).astype(np.int32))
_K1D = int((np.uint32(2654435761) * np.uint32(2)).astype(np.int32))
_K2D = int((np.uint32(805459861) * np.uint32(2)).astype(np.int32))

_NC = 2
_NS = 16
_NW = _NC * _NS
_LANES = 16
_CH = 16
_NLOCAL = 3
_LOCAL_ROWS = _OFFS[_NLOCAL]
_CELLS = 27
_CPAD = 32
_ROWW = 128


def _sc_encode(cxs, cys, czs, t0, t1, N):
    per_w = N // _NW
    n_chunks = per_w // _CH
    mesh = plsc.VectorSubcoreMesh(core_axis_name="c", subcore_axis_name="s")

    @functools.partial(
        pl.kernel,
        out_type=jax.ShapeDtypeStruct((N * 256,), jnp.float32),
        mesh=mesh,
        scratch_types=[
            pltpu.VMEM((_LOCAL_ROWS,), jnp.float32),
            pltpu.VMEM((_LOCAL_ROWS,), jnp.float32),
            pltpu.VMEM((_LANES,), jnp.int32),
            pltpu.VMEM((_LANES,), jnp.int32),
            pltpu.VMEM((_LANES,), jnp.int32),
            pltpu.VMEM((4 * _NUM_LEVELS, _ROWW), jnp.int32),
            pltpu.VMEM((_NLOCAL * 512,), jnp.int32),
            pltpu.VMEM((_NUM_LEVELS * 512,), jnp.float32),
            pltpu.VMEM((_NUM_LEVELS * 512,), jnp.float32),
            pltpu.VMEM((_NUM_LEVELS, 128), jnp.float32),
            pltpu.VMEM((_NUM_LEVELS, 128), jnp.int32),
            pltpu.VMEM((_CH * 256,), jnp.float32),
            pltpu.SemaphoreType.DMA,
        ],
        compiler_params=pltpu.CompilerParams(needs_layout_passes=False),
    )
    def enc(cx_hbm, cy_hbm, cz_hbm, t0_hbm, t1_hbm, out_hbm,
            tbl0v, tbl1v, cvx, cvy, cvz, i0b, lidxb, cb0, cb1,
            frb, dlb, featb, sem):
        wid = lax.axis_index("s") * _NC + lax.axis_index("c")
        iota = lax.iota(jnp.int32, _LANES)
        f32 = jnp.float32
        i32 = jnp.int32

        pltpu.sync_copy(t0_hbm.at[pl.ds(0, _LOCAL_ROWS)], tbl0v)
        pltpu.sync_copy(t1_hbm.at[pl.ds(0, _LOCAL_ROWS)], tbl1v)

        def chunk_body(ci, carry):
            base = wid * per_w + ci * _CH
            pltpu.sync_copy(cx_hbm.at[pl.ds(base, _CH)], cvx)
            pltpu.sync_copy(cy_hbm.at[pl.ds(base, _CH)], cvy)
            pltpu.sync_copy(cz_hbm.at[pl.ds(base, _CH)], cvz)
            fpx0 = cvx[...].astype(f32)
            fpy0 = cvy[...].astype(f32)
            fpz0 = cvz[...].astype(f32)
            fpx1 = fpx0 + 1.0
            fpy1 = fpy0 + 1.0
            fpz1 = fpz0 + 1.0

            for l in range(_NUM_LEVELS):
                scf = float(_SCALES[l] / 512.0)
                res = _RESS[l]
                off = _OFFS[l]
                hashed = res ** 3 > _HASH_SIZE
                posx0 = fpx0 * scf + 0.5
                posy0 = fpy0 * scf + 0.5
                posz0 = fpz0 * scf + 0.5
                posx1 = fpx1 * scf + 0.5
                posy1 = fpy1 * scf + 0.5
                posz1 = fpz1 * scf + 0.5
                pgx0 = posx0.astype(i32)
                pgy0 = posy0.astype(i32)
                pgz0 = posz0.astype(i32)
                pgx1 = posx1.astype(i32)
                pgy1 = posy1.astype(i32)
                pgz1 = posz1.astype(i32)
                frb[l, pl.ds(0, 16)] = posx0 - pgx0.astype(f32)
                frb[l, pl.ds(16, 16)] = posy0 - pgy0.astype(f32)
                frb[l, pl.ds(32, 16)] = posz0 - pgz0.astype(f32)
                frb[l, pl.ds(48, 16)] = posx1 - pgx1.astype(f32)
                frb[l, pl.ds(64, 16)] = posy1 - pgy1.astype(f32)
                frb[l, pl.ds(80, 16)] = posz1 - pgz1.astype(f32)
                dlb[l, pl.ds(0, 16)] = pgx1 - pgx0
                dlb[l, pl.ds(16, 16)] = pgy1 - pgy0
                dlb[l, pl.ds(32, 16)] = pgz1 - pgz0
                if hashed:
                    xs = [pgx0, pgx0 + 1, pgx0 + 2]
                    ty_0 = pgy0 * _K1
                    tys = [ty_0, ty_0 + _K1, ty_0 + _K1D]
                    tz_0 = pgz0 * _K2
                    tzs = [tz_0, tz_0 + _K2, tz_0 + _K2D]
                else:
                    r1 = res - 1
                    xs = [jnp.minimum(pgx0 + e, r1) for e in range(3)]
                    tys = [jnp.minimum(pgy0 + e, r1) * res for e in range(3)]
                    tzs = [jnp.minimum(pgz0 + e, r1) * (res * res)
                           for e in range(3)]
                for c in range(_CPAD):
                    cc = min(c, _CELLS - 1)
                    ex, ey, ez = cc % 3, (cc // 3) % 3, cc // 9
                    if hashed:
                        ridx = ((xs[ex] ^ tys[ey] ^ tzs[ez]) & _MASK) + off
                    else:
                        ridx = xs[ex] + tys[ey] + tzs[ez] + off
                    if l < _NLOCAL:
                        lidxb[pl.ds(l * 512 + c * 16, 16)] = ridx
                    else:
                        i0b[4 * l + c // 8, pl.ds((c % 8) * 16, 16)] = ridx
                if l >= _NLOCAL:
                    for t in range(4):
                        r = 4 * l + t
                        pltpu.async_copy(
                            t0_hbm.at[i0b.at[r]],
                            cb0.at[pl.ds(r * _ROWW, _ROWW)], sem)
                        pltpu.async_copy(
                            t1_hbm.at[i0b.at[r]],
                            cb1.at[pl.ds(r * _ROWW, _ROWW)], sem)

            def drain_body(r, carry2):
                pltpu.make_async_copy(
                    t0_hbm.at[i0b.at[r]],
                    cb0.at[pl.ds(r * _ROWW, _ROWW)], sem).wait()
                pltpu.make_async_copy(
                    t1_hbm.at[i0b.at[r]],
                    cb1.at[pl.ds(r * _ROWW, _ROWW)], sem).wait()
                return carry2

            lax.fori_loop(4 * _NLOCAL, 4 * _NUM_LEVELS, drain_body, 0)

            def acc_level(s, local):
                frx0 = frb[s, pl.ds(0, 16)]
                fry0 = frb[s, pl.ds(16, 16)]
                frz0 = frb[s, pl.ds(32, 16)]
                frx1 = frb[s, pl.ds(48, 16)]
                fry1 = frb[s, pl.ds(64, 16)]
                frz1 = frb[s, pl.ds(80, 16)]
                dx = dlb[s, pl.ds(0, 16)]
                dy3 = dlb[s, pl.ds(16, 16)] * 3
                dz9 = dlb[s, pl.ds(32, 16)] * 9
                gx0 = 1.0 - frx0
                gy0 = 1.0 - fry0
                gz0 = 1.0 - frz0
                gx1 = 1.0 - frx1
                gy1 = 1.0 - fry1
                gz1 = 1.0 - frz1
                cbase = jnp.full((_LANES,), s * 512, i32) + iota
                for k in range(8):
                    kx, ky, kz = k & 1, (k >> 1) & 1, k >> 2
                    bk = None
                    for term, flag in ((dx, kx), (dy3, ky), (dz9, kz)):
                        if flag:
                            bk = term if bk is None else bk + term
                    pk = cbase if bk is None else cbase + (bk << 4)
                    fx = frx1 if kx else frx0
                    fy = fry1 if ky else fry0
                    fz = frz1 if kz else frz0
                    gx = gx1 if kx else gx0
                    gy = gy1 if ky else gy0
                    gz = gz1 if kz else gz0
                    w00 = gy * gz
                    w10 = fy * gz
                    w01 = gy * fz
                    w11 = fy * fz
                    acc0 = jnp.zeros((_LANES,), f32)
                    acc1 = jnp.zeros((_LANES,), f32)
                    for d in range(8):
                        ddx, ddy, ddz = d & 1, (d >> 1) & 1, d >> 2
                        cd = (ddx + 3 * ddy + 9 * ddz) * 16
                        pos = pk + cd
                        wyz = (w11 if ddz else w10) if ddy else (w01 if ddz else w00)
                        wv = (fx if ddx else gx) * wyz
                        if local:
                            ridx = plsc.load_gather(lidxb, [pos])
                            v0 = plsc.load_gather(tbl0v, [ridx])
                            v1 = plsc.load_gather(tbl1v, [ridx])
                        else:
                            v0 = plsc.load_gather(cb0, [pos])
                            v1 = plsc.load_gather(cb1, [pos])
                        acc0 = acc0 + wv * v0
                        acc1 = acc1 + wv * v1
                    p0 = iota * 256 + (2 * s + k * 32)
                    plsc.store_scatter(featb, [p0], acc0)
                    plsc.store_scatter(featb, [p0 + 1], acc1)

            def acc_local(s, carry2):
                acc_level(s, True)
                return carry2

            def acc_stream(s, carry2):
                acc_level(s, False)
                return carry2

            lax.fori_loop(0, _NLOCAL, acc_local, 0)
            lax.fori_loop(_NLOCAL, _NUM_LEVELS, acc_stream, 0)
            pltpu.sync_copy(featb, out_hbm.at[pl.ds(base * 256, _CH * 256)])
            return carry

        lax.fori_loop(0, n_chunks, chunk_body, 0)

    return enc(cxs, cys, czs, t0, t1)


def _tc_mlp(feats2d, wk, W0b, W1b, N):
    Bn = 1024
    grid = (N // Bn,)

    def body(f_ref, w_ref, w0_ref, w1_ref, o_ref):
        x = f_ref[...]
        h = jnp.maximum(
            jnp.dot(x, w0_ref[...], preferred_element_type=jnp.float32), 0.0)
        y = jnp.dot(h, w1_ref[...], preferred_element_type=jnp.float32)
        y3 = y.reshape(Bn, 2, 32)
        wv = w_ref[...]
        acc = None
        for k in range(8):
            a, b = k // 4, k % 4
            term = wv[k][:, None] * y3[:, a, b * 8:(b + 1) * 8]
            acc = term if acc is None else acc + term
        o_ref[...] = acc

    return pl.pallas_call(
        body,
        grid=grid,
        in_specs=[
            pl.BlockSpec((2 * Bn, 128), lambda i: (i, 0)),
            pl.BlockSpec((8, Bn), lambda i: (0, i)),
            pl.BlockSpec((128, 256), lambda i: (0, 0)),
            pl.BlockSpec((256, 32), lambda i: (0, 0)),
        ],
        out_specs=pl.BlockSpec((Bn, 8), lambda i: (i, 0)),
        out_shape=jax.ShapeDtypeStruct((N, 8), jnp.float32),
    )(feats2d, wk, W0b, W1b)


def kernel(xyz, bound, table, W0, W1):
    N = xyz.shape[0]
    b = jnp.float32(bound)
    x = (xyz + b) / (2.0 * b)
    coords = x * float(_GRID_RES)
    c0 = jnp.clip(jnp.floor(coords), 0, _GRID_RES - 1).astype(jnp.int32)
    frac = coords - c0.astype(jnp.float32)
    u, v, w = frac[:, 0], frac[:, 1], frac[:, 2]
    kb = np.arange(8)
    kx = (kb & 1).astype(bool)[:, None]
    ky = ((kb >> 1) & 1).astype(bool)[:, None]
    kz = ((kb >> 2) & 1).astype(bool)[:, None]
    wk = (jnp.where(kx, u, 1 - u) * jnp.where(ky, v, 1 - v)
          * jnp.where(kz, w, 1 - w)).astype(jnp.float32)
    t0 = table[:, 0]
    t1 = table[:, 1]
    feats = _sc_encode(c0[:, 0], c0[:, 1], c0[:, 2], t0, t1, N)
    W0b = jnp.zeros((128, 256), jnp.float32)
    W1b = jnp.zeros((256, 32), jnp.float32)
    for kk in range(4):
        W0b = W0b.at[kk * 32:(kk + 1) * 32, kk * 64:(kk + 1) * 64].set(W0)
        W1b = W1b.at[kk * 64:(kk + 1) * 64, kk * 8:(kk + 1) * 8].set(W1)
    return _tc_mlp(feats.reshape(2 * N, 128), wk, W0b, W1b, N)
